# baseline (speedup 1.0000x reference)
import jax
import jax.numpy as jnp
from jax import lax
from jax.experimental import pallas as pl
from jax.experimental.pallas import tpu as pltpu
from jax.experimental.pallas import tpu_sc as plsc

VOCAB = 1000
BATCH = 16384

_info = plsc.get_sparse_core_info()
_NC, _NS, _L = _info.num_cores, _info.num_subcores, _info.num_lanes
_NW = _NC * _NS
_ROWS_PER_W = BATCH // _NW
CHUNK = 32
NBUF = 2
_NCHUNK = _ROWS_PER_W // CHUNK
_VPC = CHUNK // _L


def _sc_onehot(inp_hbm, lmap_hbm, zeros_hbm, out_hbm,
               inp_v, lmap_v, *bufs_and_sems):
    bufs = bufs_and_sems[:NBUF]
    sems = bufs_and_sems[NBUF:]
    wid = lax.axis_index("s") * _NC + lax.axis_index("c")
    base_row = wid * _ROWS_PER_W

    pltpu.sync_copy(inp_hbm.at[pl.ds(base_row, _ROWS_PER_W)], inp_v)
    pltpu.sync_copy(lmap_hbm, lmap_v)
    for b in bufs:
        pltpu.sync_copy(zeros_hbm, b)

    lane = lax.iota(jnp.int32, _L)
    ones = jnp.full((_L,), 1, jnp.int32)
    zero = jnp.full((_L,), 0, jnp.int32)
    handles = [None] * NBUF

    def chunk_ids(c, j):
        raw = inp_v[pl.ds(c * CHUNK + j * _L, _L)]
        return plsc.load_gather(lmap_v, [raw])

    for c in range(_NCHUNK):
        b = c % NBUF
        if handles[b] is not None:
            handles[b].wait()
            for j in range(_VPC):
                plsc.store_scatter(bufs[b], [lane + j * _L,
                                             chunk_ids(c - NBUF, j)], zero)
        for j in range(_VPC):
            plsc.store_scatter(bufs[b], [lane + j * _L,
                                         chunk_ids(c, j)], ones)
        handles[b] = pltpu.async_copy(
            bufs[b], out_hbm.at[pl.ds(base_row + c * CHUNK, CHUNK)], sems[b])

    for h in handles:
        h.wait()


def kernel(input, label_map):
    zeros = jnp.zeros((CHUNK, VOCAB), jnp.int32)
    run = pl.kernel(
        _sc_onehot,
        out_type=jax.ShapeDtypeStruct((BATCH, VOCAB), jnp.int32),
        mesh=plsc.VectorSubcoreMesh(core_axis_name="c", subcore_axis_name="s"),
        compiler_params=pltpu.CompilerParams(
            needs_layout_passes=False, use_tc_tiling_on_sc=True),
        scratch_types=(
            [pltpu.VMEM((_ROWS_PER_W,), jnp.int32),
             pltpu.VMEM((VOCAB,), jnp.int32)]
            + [pltpu.VMEM((CHUNK, VOCAB), jnp.int32)] * NBUF
            + [pltpu.SemaphoreType.DMA] * NBUF
        ),
    )
    return run(input, label_map, zeros)


# speedup vs baseline: 2.0080x; 2.0080x over previous
import jax
import jax.numpy as jnp
from jax import lax
from jax.experimental import pallas as pl
from jax.experimental.pallas import tpu as pltpu
from jax.experimental.pallas import tpu_sc as plsc

VOCAB = 1000
BATCH = 16384

_info = plsc.get_sparse_core_info()
_NC, _NS, _L = _info.num_cores, _info.num_subcores, _info.num_lanes
_NW = _NC * _NS
_COLS_PER_W = BATCH // _NW
CHUNK = 128
_NCHUNK = _COLS_PER_W // CHUNK
_VPC = CHUNK // _L


def _sc_onehot_t(inp_hbm, lmap_hbm, zeros_hbm, out_hbm, inp_v, lmap_v, buf):
    wid = lax.axis_index("s") * _NC + lax.axis_index("c")
    base_col = wid * _COLS_PER_W

    pltpu.sync_copy(inp_hbm.at[pl.ds(base_col, _COLS_PER_W)], inp_v)
    pltpu.sync_copy(lmap_hbm, lmap_v)
    pltpu.sync_copy(zeros_hbm, buf)

    lane = lax.iota(jnp.int32, _L)
    ones = jnp.full((_L,), 1, jnp.int32)
    zero = jnp.full((_L,), 0, jnp.int32)

    def chunk_ids(c, j):
        raw = inp_v[pl.ds(c * CHUNK + j * _L, _L)]
        return plsc.load_gather(lmap_v, [raw])

    for c in range(_NCHUNK):
        for j in range(_VPC):
            plsc.store_scatter(buf, [chunk_ids(c, j), lane + j * _L], ones)
        pltpu.sync_copy(buf, out_hbm.at[:, pl.ds(base_col + c * CHUNK, CHUNK)])
        for j in range(_VPC):
            plsc.store_scatter(buf, [chunk_ids(c, j), lane + j * _L], zero)


def kernel(input, label_map):
    zeros = jnp.zeros((VOCAB, CHUNK), jnp.int32)
    run = pl.kernel(
        _sc_onehot_t,
        out_type=jax.ShapeDtypeStruct((VOCAB, BATCH), jnp.int32),
        mesh=plsc.VectorSubcoreMesh(core_axis_name="c", subcore_axis_name="s"),
        compiler_params=pltpu.CompilerParams(
            needs_layout_passes=False, use_tc_tiling_on_sc=True),
        scratch_types=[
            pltpu.VMEM((_COLS_PER_W,), jnp.int32),
            pltpu.VMEM((VOCAB,), jnp.int32),
            pltpu.VMEM((VOCAB, CHUNK), jnp.int32),
        ],
    )
    return run(input, label_map, zeros).T


# speedup vs baseline: 2.4544x; 1.2223x over previous
import jax
import jax.numpy as jnp
from jax import lax
from jax.experimental import pallas as pl
from jax.experimental.pallas import tpu as pltpu
from jax.experimental.pallas import tpu_sc as plsc

VOCAB = 1000
BATCH = 16384

_info = plsc.get_sparse_core_info()
_NC, _NS, _L = _info.num_cores, _info.num_subcores, _info.num_lanes
_NW = _NC * _NS
_COLS_PER_W = BATCH // _NW
CHUNK = 128
_NCHUNK = _COLS_PER_W // CHUNK
_VPC = CHUNK // _L
_ZROWS = 192


def _sc_onehot_t(inp_hbm, lmap_hbm, zeros_hbm, out_hbm, inp_v, lmap_v, buf,
                 zshared):
    sid = lax.axis_index("s")
    wid = sid * _NC + lax.axis_index("c")
    base_col = wid * _COLS_PER_W

    @pl.when(sid == 0)
    def _():
        pltpu.sync_copy(zeros_hbm, zshared)

    pltpu.sync_copy(inp_hbm.at[pl.ds(base_col, _COLS_PER_W)], inp_v)
    pltpu.sync_copy(lmap_hbm, lmap_v)
    plsc.subcore_barrier()
    for r in range(0, VOCAB, _ZROWS):
        n = min(_ZROWS, VOCAB - r)
        pltpu.sync_copy(zshared.at[pl.ds(0, n)], buf.at[pl.ds(r, n)])

    lane = lax.iota(jnp.int32, _L)
    ones = jnp.full((_L,), 1, jnp.int32)
    zero = jnp.full((_L,), 0, jnp.int32)

    def chunk_ids(c, j):
        raw = inp_v[pl.ds(c * CHUNK + j * _L, _L)]
        return plsc.load_gather(lmap_v, [raw])

    for c in range(_NCHUNK):
        for j in range(_VPC):
            plsc.store_scatter(buf, [chunk_ids(c, j), lane + j * _L], ones)
        pltpu.sync_copy(buf, out_hbm.at[:, pl.ds(base_col + c * CHUNK, CHUNK)])
        for j in range(_VPC):
            plsc.store_scatter(buf, [chunk_ids(c, j), lane + j * _L], zero)


def kernel(input, label_map):
    zeros = jnp.zeros((_ZROWS, CHUNK), jnp.int32)
    run = pl.kernel(
        _sc_onehot_t,
        out_type=jax.ShapeDtypeStruct((VOCAB, BATCH), jnp.int32),
        mesh=plsc.VectorSubcoreMesh(core_axis_name="c", subcore_axis_name="s"),
        compiler_params=pltpu.CompilerParams(
            needs_layout_passes=False, use_tc_tiling_on_sc=True),
        scratch_types=[
            pltpu.VMEM((_COLS_PER_W,), jnp.int32),
            pltpu.VMEM((VOCAB,), jnp.int32),
            pltpu.VMEM((VOCAB, CHUNK), jnp.int32),
            pltpu.VMEM_SHARED((_ZROWS, CHUNK), jnp.int32),
        ],
    )
    return run(input, label_map, zeros).T
